# baseline (speedup 1.0000x reference)
import functools

import jax
import jax.numpy as jnp
from jax import lax
from jax.experimental import pallas as pl
from jax.experimental.pallas import tpu as pltpu
from jax.experimental.pallas import tpu_sc as plsc

_RATIO = 16


def _mask_body(mask_ref, gmask_ref):
    gmask_ref[...] = jnp.all(mask_ref[...], axis=2)


def _make_sc_fill(B, Sg, D):
    info = plsc.get_sparse_core_info()
    NC, NS = info.num_cores, info.num_subcores
    NW = NC * NS
    ROWS = B * Sg
    assert ROWS % NW == 0
    rpw = ROWS // NW
    K = 16
    assert rpw % K == 0
    n_chunks = rpw // K
    assert Sg % rpw == 0
    special_stride = Sg // rpw

    mesh = plsc.VectorSubcoreMesh(core_axis_name="c", subcore_axis_name="s")

    @functools.partial(
        pl.kernel,
        mesh=mesh,
        out_type=jax.ShapeDtypeStruct((ROWS * D,), jnp.float32),
        scratch_types=[
            pltpu.VMEM((K * D,), jnp.float32),
            pltpu.VMEM((D,), jnp.float32),
        ],
    )
    def fill(w_hbm, out_hbm, buf, wbuf):
        cid = lax.axis_index("c")
        sid = lax.axis_index("s")
        wid = sid * NC + cid
        base = wid * rpw

        pltpu.sync_copy(w_hbm.at[pl.ds(0, D)], buf.at[pl.ds(0, D)])

        def _rep(i, carry):
            v = buf[pl.ds(i * 16, 16)]
            for k in range(1, K):
                buf[pl.ds(k * D + i * 16, 16)] = v
            return carry

        lax.fori_loop(0, D // 16, _rep, 0)
        pltpu.sync_copy(w_hbm.at[pl.ds(D, D)], wbuf)

        is_special = lax.rem(wid, special_stride) == 0

        @pl.when(is_special)
        def _():
            pltpu.sync_copy(wbuf, out_hbm.at[pl.ds(base * D, D)])
            pltpu.sync_copy(buf.at[pl.ds(0, (K - 1) * D)],
                            out_hbm.at[pl.ds((base + 1) * D, (K - 1) * D)])

        @pl.when(jnp.logical_not(is_special))
        def _():
            pltpu.sync_copy(buf, out_hbm.at[pl.ds(base * D, K * D)])

        for c in range(1, n_chunks):
            pltpu.sync_copy(buf, out_hbm.at[pl.ds((base + c * K) * D, K * D)])

    return fill


def kernel(token_ids, padding_mask, embeds_weight):
    B, Sl = padding_mask.shape
    Sg = Sl // _RATIO
    D = embeds_weight.shape[1]

    fill = _make_sc_fill(B, Sg, D)
    emb_flat = fill(embeds_weight.reshape(2 * D))
    emb = emb_flat.reshape(B, Sg, D)

    mask3 = padding_mask.reshape(B, Sg, _RATIO)
    gmask = pl.pallas_call(
        _mask_body,
        out_shape=jax.ShapeDtypeStruct((B, Sg), jnp.bool_),
    )(mask3)
    return (emb, gmask)


# speedup vs baseline: 1.0364x; 1.0364x over previous
import functools

import jax
import jax.numpy as jnp
from jax import lax
from jax.experimental import pallas as pl
from jax.experimental.pallas import tpu as pltpu
from jax.experimental.pallas import tpu_sc as plsc

_RATIO = 16


def _mask_body(mask_ref, gmask_ref):
    gmask_ref[...] = jnp.all(mask_ref[...], axis=2)


def _make_sc_fill(B, Sg, D):
    info = plsc.get_sparse_core_info()
    NC, NS = info.num_cores, info.num_subcores
    NW = NC * NS
    ROWS = B * Sg
    assert ROWS % NW == 0
    rpw = ROWS // NW
    K = 32
    assert rpw % K == 0
    n_chunks = rpw // K
    assert Sg % rpw == 0
    special_stride = Sg // rpw

    mesh = plsc.VectorSubcoreMesh(core_axis_name="c", subcore_axis_name="s")

    @functools.partial(
        pl.kernel,
        mesh=mesh,
        out_type=jax.ShapeDtypeStruct((ROWS * D,), jnp.float32),
        scratch_types=[
            pltpu.VMEM((K * D,), jnp.float32),
            pltpu.VMEM((D,), jnp.float32),
            pltpu.SemaphoreType.DMA,
        ],
    )
    def fill(w_hbm, out_hbm, buf, wbuf, sem):
        cid = lax.axis_index("c")
        sid = lax.axis_index("s")
        wid = sid * NC + cid
        base = wid * rpw

        pltpu.sync_copy(w_hbm.at[pl.ds(0, D)], buf.at[pl.ds(0, D)])

        def _rep(i, carry):
            v = buf[pl.ds(i * 16, 16)]
            for k in range(1, K):
                buf[pl.ds(k * D + i * 16, 16)] = v
            return carry

        lax.fori_loop(0, D // 16, _rep, 0)

        is_special = lax.rem(wid, special_stride) == 0

        @pl.when(is_special)
        def _():
            pltpu.sync_copy(w_hbm.at[pl.ds(D, D)], wbuf)
            pltpu.async_copy(wbuf, out_hbm.at[pl.ds(base * D, D)], sem)
            pltpu.async_copy(buf.at[pl.ds(0, (K - 1) * D)],
                             out_hbm.at[pl.ds((base + 1) * D, (K - 1) * D)],
                             sem)

        @pl.when(jnp.logical_not(is_special))
        def _():
            pltpu.async_copy(buf, out_hbm.at[pl.ds(base * D, K * D)], sem)

        for c in range(1, n_chunks):
            pltpu.async_copy(buf, out_hbm.at[pl.ds((base + c * K) * D, K * D)],
                             sem)

        @pl.when(is_special)
        def _():
            pltpu.make_async_copy(
                wbuf, out_hbm.at[pl.ds(base * D, D)], sem).wait()
            pltpu.make_async_copy(
                buf.at[pl.ds(0, (K - 1) * D)],
                out_hbm.at[pl.ds((base + 1) * D, (K - 1) * D)], sem).wait()

        @pl.when(jnp.logical_not(is_special))
        def _():
            pltpu.make_async_copy(
                buf, out_hbm.at[pl.ds(base * D, K * D)], sem).wait()

        for c in range(1, n_chunks):
            pltpu.make_async_copy(
                buf, out_hbm.at[pl.ds((base + c * K) * D, K * D)], sem).wait()

    return fill


def kernel(token_ids, padding_mask, embeds_weight):
    B, Sl = padding_mask.shape
    Sg = Sl // _RATIO
    D = embeds_weight.shape[1]

    fill = _make_sc_fill(B, Sg, D)
    emb_flat = fill(embeds_weight.reshape(2 * D))
    emb = emb_flat.reshape(B, Sg, D)

    mask3 = padding_mask.reshape(B, Sg, _RATIO)
    gmask = pl.pallas_call(
        _mask_body,
        out_shape=jax.ShapeDtypeStruct((B, Sg), jnp.bool_),
    )(mask3)
    return (emb, gmask)


# speedup vs baseline: 1.3696x; 1.3215x over previous
import functools

import jax
import jax.numpy as jnp
from jax import lax
from jax.experimental import pallas as pl
from jax.experimental.pallas import tpu as pltpu
from jax.experimental.pallas import tpu_sc as plsc

_RATIO = 16


def _mask_body(mask_ref, gmask_ref):
    gmask_ref[...] = jnp.all(mask_ref[...], axis=2)


def _make_sc_fill(B, Sg, D):
    info = plsc.get_sparse_core_info()
    NC, NS = info.num_cores, info.num_subcores
    NW = NC * NS
    ROWS = B * Sg
    assert ROWS % NW == 0
    rpw = ROWS // NW
    K = 32
    assert rpw % K == 0
    n_chunks = rpw // K
    assert Sg % rpw == 0
    wpb = Sg // rpw

    mesh = plsc.VectorSubcoreMesh(core_axis_name="c", subcore_axis_name="s")

    @functools.partial(
        pl.kernel,
        mesh=mesh,
        out_type=jax.ShapeDtypeStruct((B, Sg, D), jnp.float32),
        scratch_types=[
            pltpu.VMEM((K + 8, D), jnp.float32),
            pltpu.SemaphoreType.DMA,
        ],
    )
    def fill(w_hbm, out_hbm, buf, sem):
        cid = lax.axis_index("c")
        sid = lax.axis_index("s")
        wid = sid * NC + cid
        b = lax.div(wid, wpb)
        sg0 = pl.multiple_of(lax.rem(wid, wpb) * rpw, rpw)

        pltpu.sync_copy(w_hbm.at[pl.ds(0, D)], buf.at[0])

        def _rep(i, carry):
            v = buf[0, pl.ds(i * 16, 16)]
            for k in range(1, K + 8):
                buf[k, pl.ds(i * 16, 16)] = v
            return carry

        lax.fori_loop(0, D // 16, _rep, 0)

        @pl.when(lax.rem(wid, wpb) == 0)
        def _():
            pltpu.sync_copy(w_hbm.at[pl.ds(D, D)], buf.at[0])

        pltpu.async_copy(buf.at[pl.ds(0, K), :],
                         out_hbm.at[b, pl.ds(sg0, K), :], sem)
        for c in range(1, n_chunks):
            pltpu.async_copy(buf.at[pl.ds(8, K), :],
                             out_hbm.at[b, pl.ds(sg0 + c * K, K), :], sem)

        pltpu.make_async_copy(buf.at[pl.ds(0, K), :],
                              out_hbm.at[b, pl.ds(sg0, K), :], sem).wait()
        for c in range(1, n_chunks):
            pltpu.make_async_copy(
                buf.at[pl.ds(8, K), :],
                out_hbm.at[b, pl.ds(sg0 + c * K, K), :], sem).wait()

    return fill


def kernel(token_ids, padding_mask, embeds_weight):
    B, Sl = padding_mask.shape
    Sg = Sl // _RATIO
    D = embeds_weight.shape[1]

    fill = _make_sc_fill(B, Sg, D)
    emb = fill(embeds_weight.reshape(2 * D))

    mask3 = padding_mask.reshape(B, Sg, _RATIO)
    gmask = pl.pallas_call(
        _mask_body,
        out_shape=jax.ShapeDtypeStruct((B, Sg), jnp.bool_),
    )(mask3)
    return (emb, gmask)


# speedup vs baseline: 2.8996x; 2.1171x over previous
import jax
import jax.numpy as jnp
from jax.experimental import pallas as pl
from jax.experimental.pallas import tpu as pltpu

_RATIO = 16


def _body(mask_ref, w_ref, emb_ref, gmask_ref):
    B, Sgb, D = emb_ref.shape
    w0 = w_ref[0, :]
    emb_ref[...] = jnp.broadcast_to(w0[None, None, :], (B, Sgb, D))

    @pl.when(pl.program_id(0) == 0)
    def _():
        emb_ref[:, 0, :] = jnp.broadcast_to(w_ref[1, :][None, :], (B, D))
        gmask_ref[...] = jnp.all(mask_ref[...], axis=2)


def kernel(token_ids, padding_mask, embeds_weight):
    B, Sl = padding_mask.shape
    Sg = Sl // _RATIO
    D = embeds_weight.shape[1]
    mask3 = padding_mask.reshape(B, Sg, _RATIO)
    sgb = 32
    grid = (Sg // sgb,)
    emb, gmask = pl.pallas_call(
        _body,
        grid=grid,
        compiler_params=pltpu.CompilerParams(
            dimension_semantics=("arbitrary",)),
        in_specs=[
            pl.BlockSpec((B, Sg, _RATIO), lambda i: (0, 0, 0)),
            pl.BlockSpec((2, D), lambda i: (0, 0)),
        ],
        out_specs=(
            pl.BlockSpec((B, sgb, D), lambda i: (0, i, 0)),
            pl.BlockSpec((B, Sg), lambda i: (0, 0)),
        ),
        out_shape=(
            jax.ShapeDtypeStruct((B, Sg, D), embeds_weight.dtype),
            jax.ShapeDtypeStruct((B, Sg), jnp.bool_),
        ),
    )(mask3, embeds_weight)
    return (emb, gmask)


# speedup vs baseline: 3.9572x; 1.3647x over previous
import jax
import jax.numpy as jnp
from jax.experimental import pallas as pl

_RATIO = 16


def _body(mask_ref, w_ref, emb_ref, gmask_ref):
    B, Sg, D = emb_ref.shape
    w0 = w_ref[0, :]
    w1 = w_ref[1, :]
    emb_ref[...] = jnp.broadcast_to(w0[None, None, :], (B, Sg, D))
    emb_ref[:, 0, :] = jnp.broadcast_to(w1[None, :], (B, D))
    gmask_ref[...] = jnp.all(mask_ref[...], axis=2)


def kernel(token_ids, padding_mask, embeds_weight):
    B, Sl = padding_mask.shape
    Sg = Sl // _RATIO
    D = embeds_weight.shape[1]
    mask3 = padding_mask.reshape(B, Sg, _RATIO)
    emb, gmask = pl.pallas_call(
        _body,
        out_shape=(
            jax.ShapeDtypeStruct((B, Sg, D), embeds_weight.dtype),
            jax.ShapeDtypeStruct((B, Sg), jnp.bool_),
        ),
    )(mask3, embeds_weight)
    return (emb, gmask)


# speedup vs baseline: 4.5189x; 1.1419x over previous
import jax
import jax.numpy as jnp
from jax.experimental import pallas as pl

_RATIO = 16


def _body(mask_ref, w_ref, emb_ref, gmask_ref):
    B, Sg, D = emb_ref.shape
    w0 = w_ref[0, :]
    w1 = w_ref[1, :]
    emb_ref[...] = jnp.broadcast_to(w0[None, None, :], (B, Sg, D))
    emb_ref[:, 0, :] = jnp.broadcast_to(w1[None, :], (B, D))

    Bm, Sl = mask_ref.shape
    L = 128
    G = L // _RATIO
    mf = mask_ref[...].astype(jnp.float32).reshape(Bm * Sl // L, L)
    sel = (jax.lax.broadcasted_iota(jnp.int32, (L, G), 0) // _RATIO
           == jax.lax.broadcasted_iota(jnp.int32, (L, G), 1)
           ).astype(jnp.float32)
    s = jax.lax.dot_general(mf, sel, (((1,), (0,)), ((), ())),
                            preferred_element_type=jnp.float32)
    gmask_ref[...] = jnp.where(s == float(_RATIO), 1, 0).astype(jnp.int32)


def kernel(token_ids, padding_mask, embeds_weight):
    B, Sl = padding_mask.shape
    Sg = Sl // _RATIO
    D = embeds_weight.shape[1]
    mask2 = padding_mask.astype(jnp.int32)
    emb, gmask = pl.pallas_call(
        _body,
        out_shape=(
            jax.ShapeDtypeStruct((B, Sg, D), embeds_weight.dtype),
            jax.ShapeDtypeStruct((B * Sl // 128, 128 // _RATIO), jnp.int32),
        ),
    )(mask2, embeds_weight)
    return (emb, gmask.reshape(B, Sg) != 0)
